# Initial kernel scaffold; baseline (speedup 1.0000x reference)
import functools

import jax
import jax.numpy as jnp
import numpy as np
from jax import lax
from jax.experimental import pallas as pl
from jax.experimental.pallas import tpu as pltpu

N = 10000
E = 320000
NODE_DIM = 128
EDGE_DIM = 16
EMBED = 64
HEADS = 8
HEAD_DIM = EMBED // HEADS
SCALE = HEAD_DIM ** -0.5

NB = 10
BN = N // NB
EB = 80
BE = E // EB

ACC_ROWS = 80


def _ln(z, g, b):
    m = jnp.mean(z, axis=-1, keepdims=True)
    v = jnp.mean((z - m) * (z - m), axis=-1, keepdims=True)
    return (z - m) * lax.rsqrt(v + 1e-5) * g + b


def _mm(a, b):
    return jax.lax.dot_general(a, b, (((1,), (0,)), ((), ())),
                               preferred_element_type=jnp.float32)


def _tr(ident, a):
    return jax.lax.dot_general(ident, a, (((1,), (1,)), ((), ())),
                               preferred_element_type=jnp.float32)


def _node_kernel(xb, w, center_o, cn_o, hx_o, qn_o):
    x = xb[:]
    h = jax.nn.relu(_ln(_mm(x, w['c_w1']) + w['c_b1'], w['c_g1'], w['c_be1']))
    h = jax.nn.relu(_ln(_mm(h, w['c_w2']) + w['c_b2'], w['c_g2'], w['c_be2']))
    center = _ln(_mm(h, w['c_w3']) + w['c_b3'], w['c_g3'], w['c_be3'])
    cn = _ln(center, w['ln1_g'], w['ln1_b'])
    hx = jax.nn.relu(_ln(_mm(x, w['nx_w1']) + w['nx_b1'], w['nx_g1'], w['nx_be1']))
    hx = _mm(hx, w['nx_w2']) + w['nx_b2']
    center_o[:] = center
    cn_o[:] = cn
    hx_o[:] = hx
    qn_o[:] = _mm(cn, w['q_w']) + w['q_b']


def _edge_kernel(hxe_b, qe_b, ea_b, w, vt_o, at_o, cmax_o):
    i = pl.program_id(0)
    ea = ea_b[:]
    he = jax.nn.relu(_ln(_mm(ea, w['ne_w1']) + w['ne_b1'], w['ne_g1'], w['ne_be1']))
    he = _mm(he, w['ne_w2']) + w['ne_b2']
    nbr = hxe_b[:] + he
    nbr = jax.nn.relu(_ln(nbr, w['na_g1'], w['na_be1']))
    nbr = _ln(_mm(nbr, w['na_w']) + w['na_b'], w['na_g2'], w['na_be2'])
    k = _mm(nbr, w['k_w']) + w['k_b']
    v = _mm(nbr, w['v_w']) + w['v_b']
    alpha = _mm(qe_b[:] * k, w['seg8']) * SCALE
    at = _tr(w['eye8'], alpha)
    vt_o[:] = _tr(w['eye64'], v)
    at_o[:] = at
    bmax = jnp.broadcast_to(jnp.max(at, axis=1, keepdims=True), (8, 128))

    @pl.when(i == 0)
    def _():
        cmax_o[:] = bmax

    @pl.when(i > 0)
    def _():
        cmax_o[:] = jnp.maximum(cmax_o[:], bmax)


def _final_kernel(acc_b, center_b, cn_b, w, out_o):
    accs = acc_b[0] + acc_b[1]
    msum_t = accs[0:64, :]
    denom_t = accs[64:72, :]
    denom_full = _mm(w['rep8'], denom_t)
    denom_full = jnp.where(denom_full == 0.0, 1.0, denom_full)
    agg = _tr(w['eyebn'], msum_t / denom_full)
    cn = cn_b[:]
    center = center_b[:]
    gate = jax.nn.sigmoid(_mm(agg, w['ih_w']) + w['ih_b'] + _mm(cn, w['hh_w']) + w['hh_b'])
    upd = agg + gate * ((_mm(cn, w['self_w']) + w['self_b']) - agg)
    center = center + _mm(upd, w['out_w']) + w['out_b']
    cn2 = _ln(center, w['ln2_g'], w['ln2_b'])
    ff = _mm(jax.nn.relu(_mm(cn2, w['m_w1']) + w['m_b1']), w['m_w2']) + w['m_b2']
    out_o[:] = center + ff


def _full(ndim):
    return pl.BlockSpec(index_map=lambda *_: tuple(0 for _ in range(ndim)))


_WMAT = ['c_w1', 'c_w2', 'c_w3', 'nx_w1', 'nx_w2', 'ne_w1', 'ne_w2',
         'na_w', 'q_w', 'k_w', 'v_w', 'self_w', 'ih_w', 'hh_w',
         'out_w', 'm_w1', 'm_w2']
_WVEC = ['c_b1', 'c_b2', 'c_b3', 'nx_b1', 'nx_b2', 'ne_b1', 'ne_b2',
         'na_b', 'q_b', 'k_b', 'v_b', 'self_b', 'ih_b', 'hh_b',
         'out_b', 'm_b1', 'm_b2',
         'c_g1', 'c_be1', 'c_g2', 'c_be2', 'c_g3', 'c_be3', 'nx_g1',
         'nx_be1', 'ne_g1', 'ne_be1', 'na_g1', 'na_be1', 'na_g2',
         'na_be2', 'ln1_g', 'ln1_b', 'ln2_g', 'ln2_b']


def _prep_weights(p):
    w = {k: p[k].T for k in _WMAT}
    w.update({k: p[k].reshape(1, -1) for k in _WVEC})
    w['seg8'] = jnp.asarray(np.repeat(np.eye(8, dtype=np.float32), 8, axis=0))
    w['rep8'] = jnp.asarray(np.repeat(np.eye(8, dtype=np.float32), 8, axis=0))
    w['eye8'] = jnp.asarray(np.eye(8, dtype=np.float32))
    w['eye64'] = jnp.asarray(np.eye(64, dtype=np.float32))
    w['eyebn'] = jnp.asarray(np.eye(BN, dtype=np.float32))
    return w


@jax.jit
def _run(x, edge_index, edge_attr, params):
    w = _prep_weights(params)
    src = edge_index[0].astype(jnp.int32)
    dst = edge_index[1].astype(jnp.int32)

    wspec = {k: _full(jnp.ndim(v)) for k, v in w.items()}

    p1_keys = ['c_w1', 'c_b1', 'c_g1', 'c_be1', 'c_w2', 'c_b2', 'c_g2', 'c_be2',
               'c_w3', 'c_b3', 'c_g3', 'c_be3', 'ln1_g', 'ln1_b',
               'nx_w1', 'nx_b1', 'nx_g1', 'nx_be1', 'nx_w2', 'nx_b2',
               'q_w', 'q_b']
    center, cn, hx, qn = pl.pallas_call(
        _node_kernel,
        grid=(NB,),
        in_specs=[pl.BlockSpec((BN, NODE_DIM), lambda i: (i, 0)),
                  {k: wspec[k] for k in p1_keys}],
        out_specs=[pl.BlockSpec((BN, EMBED), lambda i: (i, 0))] * 4,
        out_shape=[jax.ShapeDtypeStruct((N, EMBED), jnp.float32)] * 4,
    )(x, {k: w[k] for k in p1_keys})

    hx_e = jnp.take(hx, src, axis=0)
    qe = jnp.take(qn, dst, axis=0)

    p3_keys = ['ne_w1', 'ne_b1', 'ne_g1', 'ne_be1', 'ne_w2', 'ne_b2',
               'na_g1', 'na_be1', 'na_w', 'na_b', 'na_g2', 'na_be2',
               'k_w', 'k_b', 'v_w', 'v_b', 'seg8', 'eye8', 'eye64']
    vt, at, cmax = pl.pallas_call(
        _edge_kernel,
        grid=(EB,),
        in_specs=[pl.BlockSpec((BE, EMBED), lambda i: (i, 0)),
                  pl.BlockSpec((BE, EMBED), lambda i: (i, 0)),
                  pl.BlockSpec((BE, EDGE_DIM), lambda i: (i, 0)),
                  {k: wspec[k] for k in p3_keys}],
        out_specs=[pl.BlockSpec((EMBED, BE), lambda i: (0, i)),
                   pl.BlockSpec((HEADS, BE), lambda i: (0, i)),
                   pl.BlockSpec((HEADS, 128), lambda i: (0, 0))],
        out_shape=[jax.ShapeDtypeStruct((EMBED, E), jnp.float32),
                   jax.ShapeDtypeStruct((HEADS, E), jnp.float32),
                   jax.ShapeDtypeStruct((HEADS, 128), jnp.float32)],
    )(hx_e, qe, edge_attr, {k: w[k] for k in p3_keys})

    cvec = cmax[:, 0]
    ae = jnp.exp(at - cvec[:, None])
    ae64 = jnp.repeat(ae, 8, axis=0)
    msum_t = jax.ops.segment_sum((vt * ae64).T, dst, num_segments=N).T
    denom_t = jax.ops.segment_sum(ae.T, dst, num_segments=N).T
    acc = jnp.zeros((2, ACC_ROWS, N), jnp.float32)
    acc = acc.at[0, 0:64, :].set(msum_t).at[0, 64:72, :].set(denom_t)

    p5_keys = ['ih_w', 'ih_b', 'hh_w', 'hh_b', 'self_w', 'self_b',
               'out_w', 'out_b', 'ln2_g', 'ln2_b',
               'm_w1', 'm_b1', 'm_w2', 'm_b2', 'rep8', 'eyebn']
    out = pl.pallas_call(
        _final_kernel,
        grid=(NB,),
        in_specs=[pl.BlockSpec((2, ACC_ROWS, BN), lambda i: (0, 0, i)),
                  pl.BlockSpec((BN, EMBED), lambda i: (i, 0)),
                  pl.BlockSpec((BN, EMBED), lambda i: (i, 0)),
                  {k: wspec[k] for k in p5_keys}],
        out_specs=pl.BlockSpec((BN, EMBED), lambda i: (i, 0)),
        out_shape=jax.ShapeDtypeStruct((N, EMBED), jnp.float32),
    )(acc, center, cn, {k: w[k] for k in p5_keys})
    return out


def kernel(x, edge_index, edge_attr, params):
    return _run(x, edge_index, edge_attr, params)


# speedup vs baseline: 1.5948x; 1.5948x over previous
import functools

import jax
import jax.numpy as jnp
import numpy as np
from jax import lax
from jax.experimental import pallas as pl
from jax.experimental.pallas import tpu as pltpu

N = 10000
E = 320000
NODE_DIM = 128
EDGE_DIM = 16
EMBED = 64
HEADS = 8
HEAD_DIM = EMBED // HEADS
SCALE = HEAD_DIM ** -0.5

NP = 10240
NB = 10
BN = NP // NB
EB = 50
BE = E // EB

ACC_ROWS = 80


def _ln(z, g, b):
    m = jnp.mean(z, axis=-1, keepdims=True)
    v = jnp.mean((z - m) * (z - m), axis=-1, keepdims=True)
    return (z - m) * lax.rsqrt(v + 1e-5) * g + b


def _mm(a, b):
    return jax.lax.dot_general(a, b, (((1,), (0,)), ((), ())),
                               preferred_element_type=jnp.float32)


def _tr(ident, a):
    return jax.lax.dot_general(ident, a, (((1,), (1,)), ((), ())),
                               preferred_element_type=jnp.float32)


def _tr0(a, ident):
    return jax.lax.dot_general(a, ident, (((0,), (0,)), ((), ())),
                               preferred_element_type=jnp.float32)


def _node_kernel(xb, wref, center_o, cn_o, hx_o, qn_o):
    w = {k: r[...] for k, r in wref.items()}
    x = xb[:]
    h = jax.nn.relu(_ln(_mm(x, w['c_w1']) + w['c_b1'], w['c_g1'], w['c_be1']))
    h = jax.nn.relu(_ln(_mm(h, w['c_w2']) + w['c_b2'], w['c_g2'], w['c_be2']))
    center = _ln(_mm(h, w['c_w3']) + w['c_b3'], w['c_g3'], w['c_be3'])
    cn = _ln(center, w['ln1_g'], w['ln1_b'])
    hx = jax.nn.relu(_ln(_mm(x, w['nx_w1']) + w['nx_b1'], w['nx_g1'], w['nx_be1']))
    hx = _mm(hx, w['nx_w2']) + w['nx_b2']
    center_o[:] = center
    cn_o[:] = cn
    hx_o[:] = hx
    qn_o[:] = _mm(cn, w['q_w']) + w['q_b']


def _edge_kernel(hxe_b, qe_b, ea_b, wref, vt_o, at_o, cmax_o):
    w = {k: r[...] for k, r in wref.items()}
    i = pl.program_id(0)
    ea = ea_b[:]
    he = jax.nn.relu(_ln(_mm(ea, w['ne_w1']) + w['ne_b1'], w['ne_g1'], w['ne_be1']))
    he = _mm(he, w['ne_w2']) + w['ne_b2']
    nbr = hxe_b[:] + he
    nbr = jax.nn.relu(_ln(nbr, w['na_g1'], w['na_be1']))
    nbr = _ln(_mm(nbr, w['na_w']) + w['na_b'], w['na_g2'], w['na_be2'])
    k = _mm(nbr, w['k_w']) + w['k_b']
    v = _mm(nbr, w['v_w']) + w['v_b']
    alpha = _mm(qe_b[:] * k, w['seg8']) * SCALE
    at = _tr(w['eye8'], alpha)
    vt_o[:] = _tr(w['eye64'], v)
    at_o[:] = at
    bmax = jnp.broadcast_to(jnp.max(at, axis=1, keepdims=True), (8, 128))

    @pl.when(i == 0)
    def _():
        cmax_o[:] = bmax

    @pl.when(i > 0)
    def _():
        cmax_o[:] = jnp.maximum(cmax_o[:], bmax)


def _final_kernel(acc_b, center_b, cn_b, wref, out_o):
    w = {k: r[...] for k, r in wref.items()}
    accs = acc_b[0] + acc_b[1]
    msum_t = accs[0:64, :]
    denom_t = accs[64:72, :]
    denom_full = _mm(w['rep8'], denom_t)
    denom_full = jnp.where(denom_full == 0.0, 1.0, denom_full)
    agg = _tr0(msum_t / denom_full, w['eye64'])
    cn = cn_b[:]
    center = center_b[:]
    gate = jax.nn.sigmoid(_mm(agg, w['ih_w']) + w['ih_b'] + _mm(cn, w['hh_w']) + w['hh_b'])
    upd = agg + gate * ((_mm(cn, w['self_w']) + w['self_b']) - agg)
    center = center + _mm(upd, w['out_w']) + w['out_b']
    cn2 = _ln(center, w['ln2_g'], w['ln2_b'])
    ff = _mm(jax.nn.relu(_mm(cn2, w['m_w1']) + w['m_b1']), w['m_w2']) + w['m_b2']
    out_o[:] = center + ff


def _full(ndim):
    return pl.BlockSpec(index_map=lambda *_: tuple(0 for _ in range(ndim)))


_WMAT = ['c_w1', 'c_w2', 'c_w3', 'nx_w1', 'nx_w2', 'ne_w1', 'ne_w2',
         'na_w', 'q_w', 'k_w', 'v_w', 'self_w', 'ih_w', 'hh_w',
         'out_w', 'm_w1', 'm_w2']
_WVEC = ['c_b1', 'c_b2', 'c_b3', 'nx_b1', 'nx_b2', 'ne_b1', 'ne_b2',
         'na_b', 'q_b', 'k_b', 'v_b', 'self_b', 'ih_b', 'hh_b',
         'out_b', 'm_b1', 'm_b2',
         'c_g1', 'c_be1', 'c_g2', 'c_be2', 'c_g3', 'c_be3', 'nx_g1',
         'nx_be1', 'ne_g1', 'ne_be1', 'na_g1', 'na_be1', 'na_g2',
         'na_be2', 'ln1_g', 'ln1_b', 'ln2_g', 'ln2_b']


def _prep_weights(p):
    w = {k: p[k].T for k in _WMAT}
    w.update({k: p[k].reshape(1, -1) for k in _WVEC})
    w['seg8'] = jnp.asarray(np.repeat(np.eye(8, dtype=np.float32), 8, axis=0))
    w['rep8'] = jnp.asarray(np.repeat(np.eye(8, dtype=np.float32), 8, axis=0))
    w['eye8'] = jnp.asarray(np.eye(8, dtype=np.float32))
    w['eye64'] = jnp.asarray(np.eye(64, dtype=np.float32))
    return w


@jax.jit
def _run(x, edge_index, edge_attr, params):
    w = _prep_weights(params)
    xp = jnp.pad(x, ((0, NP - N), (0, 0)))
    src = edge_index[0].astype(jnp.int32)
    dst = edge_index[1].astype(jnp.int32)

    wspec = {k: _full(jnp.ndim(v)) for k, v in w.items()}

    p1_keys = ['c_w1', 'c_b1', 'c_g1', 'c_be1', 'c_w2', 'c_b2', 'c_g2', 'c_be2',
               'c_w3', 'c_b3', 'c_g3', 'c_be3', 'ln1_g', 'ln1_b',
               'nx_w1', 'nx_b1', 'nx_g1', 'nx_be1', 'nx_w2', 'nx_b2',
               'q_w', 'q_b']
    center, cn, hx, qn = pl.pallas_call(
        _node_kernel,
        grid=(NB,),
        in_specs=[pl.BlockSpec((BN, NODE_DIM), lambda i: (i, 0)),
                  {k: wspec[k] for k in p1_keys}],
        out_specs=[pl.BlockSpec((BN, EMBED), lambda i: (i, 0))] * 4,
        out_shape=[jax.ShapeDtypeStruct((NP, EMBED), jnp.float32)] * 4,
    )(xp, {k: w[k] for k in p1_keys})

    hx_e = jnp.take(hx, src, axis=0)
    qe = jnp.take(qn, dst, axis=0)

    p3_keys = ['ne_w1', 'ne_b1', 'ne_g1', 'ne_be1', 'ne_w2', 'ne_b2',
               'na_g1', 'na_be1', 'na_w', 'na_b', 'na_g2', 'na_be2',
               'k_w', 'k_b', 'v_w', 'v_b', 'seg8', 'eye8', 'eye64']
    vt, at, cmax = pl.pallas_call(
        _edge_kernel,
        grid=(EB,),
        in_specs=[pl.BlockSpec((BE, EMBED), lambda i: (i, 0)),
                  pl.BlockSpec((BE, EMBED), lambda i: (i, 0)),
                  pl.BlockSpec((BE, EDGE_DIM), lambda i: (i, 0)),
                  {k: wspec[k] for k in p3_keys}],
        out_specs=[pl.BlockSpec((EMBED, BE), lambda i: (0, i)),
                   pl.BlockSpec((HEADS, BE), lambda i: (0, i)),
                   pl.BlockSpec((HEADS, 128), lambda i: (0, 0))],
        out_shape=[jax.ShapeDtypeStruct((EMBED, E), jnp.float32),
                   jax.ShapeDtypeStruct((HEADS, E), jnp.float32),
                   jax.ShapeDtypeStruct((HEADS, 128), jnp.float32)],
    )(hx_e, qe, edge_attr, {k: w[k] for k in p3_keys})

    cvec = cmax[:, 0]
    ae = jnp.exp(at - cvec[:, None])
    ae64 = jnp.repeat(ae, 8, axis=0)
    msum_t = jax.ops.segment_sum((vt * ae64).T, dst, num_segments=NP).T
    denom_t = jax.ops.segment_sum(ae.T, dst, num_segments=NP).T
    acc = jnp.zeros((2, ACC_ROWS, NP), jnp.float32)
    acc = acc.at[0, 0:64, :].set(msum_t).at[0, 64:72, :].set(denom_t)

    p5_keys = ['ih_w', 'ih_b', 'hh_w', 'hh_b', 'self_w', 'self_b',
               'out_w', 'out_b', 'ln2_g', 'ln2_b',
               'm_w1', 'm_b1', 'm_w2', 'm_b2', 'rep8', 'eye64']
    out = pl.pallas_call(
        _final_kernel,
        grid=(NB,),
        in_specs=[pl.BlockSpec((2, ACC_ROWS, BN), lambda i: (0, 0, i)),
                  pl.BlockSpec((BN, EMBED), lambda i: (i, 0)),
                  pl.BlockSpec((BN, EMBED), lambda i: (i, 0)),
                  {k: wspec[k] for k in p5_keys}],
        out_specs=pl.BlockSpec((BN, EMBED), lambda i: (i, 0)),
        out_shape=jax.ShapeDtypeStruct((NP, EMBED), jnp.float32),
    )(acc, center, cn, {k: w[k] for k in p5_keys})
    return out[:N]


def kernel(x, edge_index, edge_attr, params):
    return _run(x, edge_index, edge_attr, params)


# speedup vs baseline: 2.4040x; 1.5074x over previous
import functools

import jax
import jax.numpy as jnp
import numpy as np
from jax import lax
from jax.experimental import pallas as pl
from jax.experimental.pallas import tpu as pltpu
from jax.experimental.pallas import tpu_sc as plsc

N = 10000
E = 320000
NODE_DIM = 128
EDGE_DIM = 16
EMBED = 64
HEADS = 8
HEAD_DIM = EMBED // HEADS
SCALE = HEAD_DIM ** -0.5

NP = 10240
NB = 10
BN = NP // NB
EB = 50
BE = E // EB

ACC_ROWS = 80


def _ln(z, g, b):
    m = jnp.mean(z, axis=-1, keepdims=True)
    v = jnp.mean((z - m) * (z - m), axis=-1, keepdims=True)
    return (z - m) * lax.rsqrt(v + 1e-5) * g + b


def _mm(a, b):
    return jax.lax.dot_general(a, b, (((1,), (0,)), ((), ())),
                               preferred_element_type=jnp.float32)


def _tr(ident, a):
    return jax.lax.dot_general(ident, a, (((1,), (1,)), ((), ())),
                               preferred_element_type=jnp.float32)


def _tr0(a, ident):
    return jax.lax.dot_general(a, ident, (((0,), (0,)), ((), ())),
                               preferred_element_type=jnp.float32)


def _node_kernel(xb, wref, center_o, cn_o, hxq_o):
    w = {k: r[...] for k, r in wref.items()}
    x = xb[:]
    h = jax.nn.relu(_ln(_mm(x, w['c_w1']) + w['c_b1'], w['c_g1'], w['c_be1']))
    h = jax.nn.relu(_ln(_mm(h, w['c_w2']) + w['c_b2'], w['c_g2'], w['c_be2']))
    center = _ln(_mm(h, w['c_w3']) + w['c_b3'], w['c_g3'], w['c_be3'])
    cn = _ln(center, w['ln1_g'], w['ln1_b'])
    hx = jax.nn.relu(_ln(_mm(x, w['nx_w1']) + w['nx_b1'], w['nx_g1'], w['nx_be1']))
    hx = _mm(hx, w['nx_w2']) + w['nx_b2']
    qn = _mm(cn, w['q_w']) + w['q_b']
    center_o[:] = center
    cn_o[:] = cn
    hxq_o[:] = jnp.concatenate([hx, qn], axis=1)


GW = 32
GC = 100
GPW = E // GW
GCH = GPW // GC
GBUF = 400
GGRP = GPW // GBUF
GPG = GBUF // GC


def _gather_body(hxq_hbm, src_hbm, dst_hbm, hxe_hbm, qe_hbm,
                 idxs_v, idxd_v, rows_hx, rows_q, sem):
    cid = lax.axis_index("c")
    sid = lax.axis_index("s")
    wid = sid * 2 + cid
    pltpu.sync_copy(src_hbm.at[wid], idxs_v)
    pltpu.sync_copy(dst_hbm.at[wid], idxd_v)

    def group(g, _):
        cps = []
        for t in range(GPG):
            r = g * GPG + t
            cps.append(pltpu.async_copy(hxq_hbm.at[idxs_v.at[r]],
                                        rows_hx.at[pl.ds(t * GC, GC)], sem))
            cps.append(pltpu.async_copy(hxq_hbm.at[idxd_v.at[r]],
                                        rows_q.at[pl.ds(t * GC, GC)], sem))
        for cp in cps:
            cp.wait()
        obase = pl.multiple_of(wid * GPW + g * GBUF, 8)
        pltpu.sync_copy(rows_hx, hxe_hbm.at[pl.ds(obase, GBUF)])
        pltpu.sync_copy(rows_q, qe_hbm.at[pl.ds(obase, GBUF)])
        return 0

    lax.fori_loop(0, GGRP, group, 0)


def _sc_gather(hxq, src, dst):
    src3d = src.reshape(GW, GCH, GC)
    dst3d = dst.reshape(GW, GCH, GC)
    return pl.kernel(
        _gather_body,
        out_type=[jax.ShapeDtypeStruct((E, 2 * EMBED), jnp.float32),
                  jax.ShapeDtypeStruct((E, 2 * EMBED), jnp.float32)],
        mesh=plsc.VectorSubcoreMesh(core_axis_name="c", subcore_axis_name="s"),
        scratch_types=[pltpu.VMEM((GCH, GC), jnp.int32),
                       pltpu.VMEM((GCH, GC), jnp.int32),
                       pltpu.VMEM((GBUF, 2 * EMBED), jnp.float32),
                       pltpu.VMEM((GBUF, 2 * EMBED), jnp.float32),
                       pltpu.SemaphoreType.DMA],
    )(hxq, src3d, dst3d)


def _edge_kernel(hxe_b, qe_b, ea_b, wref, vt_o, at_o, cmax_o):
    w = {k: r[...] for k, r in wref.items()}
    i = pl.program_id(0)
    ea = ea_b[:]
    he = jax.nn.relu(_ln(_mm(ea, w['ne_w1']) + w['ne_b1'], w['ne_g1'], w['ne_be1']))
    he = _mm(he, w['ne_w2']) + w['ne_b2']
    nbr = hxe_b[:, 0:EMBED] + he
    nbr = jax.nn.relu(_ln(nbr, w['na_g1'], w['na_be1']))
    nbr = _ln(_mm(nbr, w['na_w']) + w['na_b'], w['na_g2'], w['na_be2'])
    k = _mm(nbr, w['k_w']) + w['k_b']
    v = _mm(nbr, w['v_w']) + w['v_b']
    alpha = _mm(qe_b[:, EMBED:2 * EMBED] * k, w['seg8']) * SCALE
    at = _tr(w['eye8'], alpha)
    vt_o[:] = _tr(w['eye64'], v)
    at_o[:] = at
    bmax = jnp.broadcast_to(jnp.max(at, axis=1, keepdims=True), (8, 128))

    @pl.when(i == 0)
    def _():
        cmax_o[:] = bmax

    @pl.when(i > 0)
    def _():
        cmax_o[:] = jnp.maximum(cmax_o[:], bmax)


def _final_kernel(acc_b, center_b, cn_b, wref, out_o):
    w = {k: r[...] for k, r in wref.items()}
    accs = acc_b[0] + acc_b[1]
    msum_t = accs[0:64, :]
    denom_t = accs[64:72, :]
    denom_full = _mm(w['rep8'], denom_t)
    denom_full = jnp.where(denom_full == 0.0, 1.0, denom_full)
    agg = _tr0(msum_t / denom_full, w['eye64'])
    cn = cn_b[:]
    center = center_b[:]
    gate = jax.nn.sigmoid(_mm(agg, w['ih_w']) + w['ih_b'] + _mm(cn, w['hh_w']) + w['hh_b'])
    upd = agg + gate * ((_mm(cn, w['self_w']) + w['self_b']) - agg)
    center = center + _mm(upd, w['out_w']) + w['out_b']
    cn2 = _ln(center, w['ln2_g'], w['ln2_b'])
    ff = _mm(jax.nn.relu(_mm(cn2, w['m_w1']) + w['m_b1']), w['m_w2']) + w['m_b2']
    out_o[:] = center + ff


def _full(ndim):
    return pl.BlockSpec(index_map=lambda *_: tuple(0 for _ in range(ndim)))


_WMAT = ['c_w1', 'c_w2', 'c_w3', 'nx_w1', 'nx_w2', 'ne_w1', 'ne_w2',
         'na_w', 'q_w', 'k_w', 'v_w', 'self_w', 'ih_w', 'hh_w',
         'out_w', 'm_w1', 'm_w2']
_WVEC = ['c_b1', 'c_b2', 'c_b3', 'nx_b1', 'nx_b2', 'ne_b1', 'ne_b2',
         'na_b', 'q_b', 'k_b', 'v_b', 'self_b', 'ih_b', 'hh_b',
         'out_b', 'm_b1', 'm_b2',
         'c_g1', 'c_be1', 'c_g2', 'c_be2', 'c_g3', 'c_be3', 'nx_g1',
         'nx_be1', 'ne_g1', 'ne_be1', 'na_g1', 'na_be1', 'na_g2',
         'na_be2', 'ln1_g', 'ln1_b', 'ln2_g', 'ln2_b']


def _prep_weights(p):
    w = {k: p[k].T for k in _WMAT}
    w.update({k: p[k].reshape(1, -1) for k in _WVEC})
    w['seg8'] = jnp.asarray(np.repeat(np.eye(8, dtype=np.float32), 8, axis=0))
    w['rep8'] = jnp.asarray(np.repeat(np.eye(8, dtype=np.float32), 8, axis=0))
    w['eye8'] = jnp.asarray(np.eye(8, dtype=np.float32))
    w['eye64'] = jnp.asarray(np.eye(64, dtype=np.float32))
    return w


@jax.jit
def _run(x, edge_index, edge_attr, params):
    w = _prep_weights(params)
    xp = jnp.pad(x, ((0, NP - N), (0, 0)))
    src = edge_index[0].astype(jnp.int32)
    dst = edge_index[1].astype(jnp.int32)

    wspec = {k: _full(jnp.ndim(v)) for k, v in w.items()}

    p1_keys = ['c_w1', 'c_b1', 'c_g1', 'c_be1', 'c_w2', 'c_b2', 'c_g2', 'c_be2',
               'c_w3', 'c_b3', 'c_g3', 'c_be3', 'ln1_g', 'ln1_b',
               'nx_w1', 'nx_b1', 'nx_g1', 'nx_be1', 'nx_w2', 'nx_b2',
               'q_w', 'q_b']
    center, cn, hxq = pl.pallas_call(
        _node_kernel,
        grid=(NB,),
        in_specs=[pl.BlockSpec((BN, NODE_DIM), lambda i: (i, 0)),
                  {k: wspec[k] for k in p1_keys}],
        out_specs=[pl.BlockSpec((BN, EMBED), lambda i: (i, 0)),
                   pl.BlockSpec((BN, EMBED), lambda i: (i, 0)),
                   pl.BlockSpec((BN, 2 * EMBED), lambda i: (i, 0))],
        out_shape=[jax.ShapeDtypeStruct((NP, EMBED), jnp.float32),
                   jax.ShapeDtypeStruct((NP, EMBED), jnp.float32),
                   jax.ShapeDtypeStruct((NP, 2 * EMBED), jnp.float32)],
    )(xp, {k: w[k] for k in p1_keys})

    hx_e, qe = _sc_gather(hxq, src, dst)

    p3_keys = ['ne_w1', 'ne_b1', 'ne_g1', 'ne_be1', 'ne_w2', 'ne_b2',
               'na_g1', 'na_be1', 'na_w', 'na_b', 'na_g2', 'na_be2',
               'k_w', 'k_b', 'v_w', 'v_b', 'seg8', 'eye8', 'eye64']
    vt, at, cmax = pl.pallas_call(
        _edge_kernel,
        grid=(EB,),
        in_specs=[pl.BlockSpec((BE, 2 * EMBED), lambda i: (i, 0)),
                  pl.BlockSpec((BE, 2 * EMBED), lambda i: (i, 0)),
                  pl.BlockSpec((BE, EDGE_DIM), lambda i: (i, 0)),
                  {k: wspec[k] for k in p3_keys}],
        out_specs=[pl.BlockSpec((EMBED, BE), lambda i: (0, i)),
                   pl.BlockSpec((HEADS, BE), lambda i: (0, i)),
                   pl.BlockSpec((HEADS, 128), lambda i: (0, 0))],
        out_shape=[jax.ShapeDtypeStruct((EMBED, E), jnp.float32),
                   jax.ShapeDtypeStruct((HEADS, E), jnp.float32),
                   jax.ShapeDtypeStruct((HEADS, 128), jnp.float32)],
    )(hx_e, qe, edge_attr, {k: w[k] for k in p3_keys})

    cvec = cmax[:, 0]
    ae = jnp.exp(at - cvec[:, None])
    ae64 = jnp.repeat(ae, 8, axis=0)
    msum_t = jax.ops.segment_sum((vt * ae64).T, dst, num_segments=NP).T
    denom_t = jax.ops.segment_sum(ae.T, dst, num_segments=NP).T
    acc = jnp.zeros((2, ACC_ROWS, NP), jnp.float32)
    acc = acc.at[0, 0:64, :].set(msum_t).at[0, 64:72, :].set(denom_t)

    p5_keys = ['ih_w', 'ih_b', 'hh_w', 'hh_b', 'self_w', 'self_b',
               'out_w', 'out_b', 'ln2_g', 'ln2_b',
               'm_w1', 'm_b1', 'm_w2', 'm_b2', 'rep8', 'eye64']
    out = pl.pallas_call(
        _final_kernel,
        grid=(NB,),
        in_specs=[pl.BlockSpec((2, ACC_ROWS, BN), lambda i: (0, 0, i)),
                  pl.BlockSpec((BN, EMBED), lambda i: (i, 0)),
                  pl.BlockSpec((BN, EMBED), lambda i: (i, 0)),
                  {k: wspec[k] for k in p5_keys}],
        out_specs=pl.BlockSpec((BN, EMBED), lambda i: (i, 0)),
        out_shape=jax.ShapeDtypeStruct((NP, EMBED), jnp.float32),
    )(acc, center, cn, {k: w[k] for k in p5_keys})
    return out[:N]


def kernel(x, edge_index, edge_attr, params):
    return _run(x, edge_index, edge_attr, params)


# speedup vs baseline: 3.2050x; 1.3332x over previous
import functools

import jax
import jax.numpy as jnp
import numpy as np
from jax import lax
from jax.experimental import pallas as pl
from jax.experimental.pallas import tpu as pltpu
from jax.experimental.pallas import tpu_sc as plsc

N = 10000
E = 320000
NODE_DIM = 128
EDGE_DIM = 16
EMBED = 64
HEADS = 8
HEAD_DIM = EMBED // HEADS
SCALE = HEAD_DIM ** -0.5

NP = 10240
NB = 10
BN = NP // NB
EB = 50
BE = E // EB

MW = 80


def _ln(z, g, b):
    m = jnp.mean(z, axis=-1, keepdims=True)
    v = jnp.mean((z - m) * (z - m), axis=-1, keepdims=True)
    return (z - m) * lax.rsqrt(v + 1e-5) * g + b


def _mm(a, b):
    return jax.lax.dot_general(a, b, (((1,), (0,)), ((), ())),
                               preferred_element_type=jnp.float32)


def _tr(ident, a):
    return jax.lax.dot_general(ident, a, (((1,), (1,)), ((), ())),
                               preferred_element_type=jnp.float32)


def _tr0(a, ident):
    return jax.lax.dot_general(a, ident, (((0,), (0,)), ((), ())),
                               preferred_element_type=jnp.float32)


def _node_kernel(xb, wref, center_o, cn_o, hxq_o):
    w = {k: r[...] for k, r in wref.items()}
    x = xb[:]
    h = jax.nn.relu(_ln(_mm(x, w['c_w1']) + w['c_b1'], w['c_g1'], w['c_be1']))
    h = jax.nn.relu(_ln(_mm(h, w['c_w2']) + w['c_b2'], w['c_g2'], w['c_be2']))
    center = _ln(_mm(h, w['c_w3']) + w['c_b3'], w['c_g3'], w['c_be3'])
    cn = _ln(center, w['ln1_g'], w['ln1_b'])
    hx = jax.nn.relu(_ln(_mm(x, w['nx_w1']) + w['nx_b1'], w['nx_g1'], w['nx_be1']))
    hx = _mm(hx, w['nx_w2']) + w['nx_b2']
    qn = _mm(cn, w['q_w']) + w['q_b']
    center_o[:] = center
    cn_o[:] = cn
    hxq_o[:] = jnp.concatenate([hx, qn], axis=1)


GW = 32
GC = 100
GPW = E // GW
GCH = GPW // GC
GBUF = 400
GGRP = GPW // GBUF
GPG = GBUF // GC


def _gather_body(hxq_hbm, src_hbm, dst_hbm, hxe_hbm, qe_hbm,
                 idxs_v, idxd_v, rows_hx, rows_q, sem):
    cid = lax.axis_index("c")
    sid = lax.axis_index("s")
    wid = sid * 2 + cid
    pltpu.sync_copy(src_hbm.at[wid], idxs_v)
    pltpu.sync_copy(dst_hbm.at[wid], idxd_v)

    def group(g, _):
        cps = []
        for t in range(GPG):
            r = g * GPG + t
            cps.append(pltpu.async_copy(hxq_hbm.at[idxs_v.at[r]],
                                        rows_hx.at[pl.ds(t * GC, GC)], sem))
            cps.append(pltpu.async_copy(hxq_hbm.at[idxd_v.at[r]],
                                        rows_q.at[pl.ds(t * GC, GC)], sem))
        for cp in cps:
            cp.wait()
        obase = pl.multiple_of(wid * GPW + g * GBUF, 8)
        pltpu.sync_copy(rows_hx, hxe_hbm.at[pl.ds(obase, GBUF)])
        pltpu.sync_copy(rows_q, qe_hbm.at[pl.ds(obase, GBUF)])
        return 0

    lax.fori_loop(0, GGRP, group, 0)


def _sc_gather(hxq, src, dst):
    src3d = src.reshape(GW, GCH, GC)
    dst3d = dst.reshape(GW, GCH, GC)
    return pl.kernel(
        _gather_body,
        out_type=[jax.ShapeDtypeStruct((E, 2 * EMBED), jnp.float32),
                  jax.ShapeDtypeStruct((E, 2 * EMBED), jnp.float32)],
        mesh=plsc.VectorSubcoreMesh(core_axis_name="c", subcore_axis_name="s"),
        scratch_types=[pltpu.VMEM((GCH, GC), jnp.int32),
                       pltpu.VMEM((GCH, GC), jnp.int32),
                       pltpu.VMEM((GBUF, 2 * EMBED), jnp.float32),
                       pltpu.VMEM((GBUF, 2 * EMBED), jnp.float32),
                       pltpu.SemaphoreType.DMA],
    )(hxq, src3d, dst3d)


def _edge_kernel(hxe_b, qe_b, ea_b, wref, msg_o, cmax_o):
    w = {k: r[...] for k, r in wref.items()}
    i = pl.program_id(0)
    ea = ea_b[:]
    he = jax.nn.relu(_ln(_mm(ea, w['ne_w1']) + w['ne_b1'], w['ne_g1'], w['ne_be1']))
    he = _mm(he, w['ne_w2']) + w['ne_b2']
    nbr = hxe_b[:, 0:EMBED] + he
    nbr = jax.nn.relu(_ln(nbr, w['na_g1'], w['na_be1']))
    nbr = _ln(_mm(nbr, w['na_w']) + w['na_b'], w['na_g2'], w['na_be2'])
    k = _mm(nbr, w['k_w']) + w['k_b']
    v = _mm(nbr, w['v_w']) + w['v_b']
    alpha = _mm(qe_b[:, EMBED:2 * EMBED] * k, w['seg8']) * SCALE
    msg_o[:] = jnp.concatenate(
        [v, alpha, jnp.zeros((BE, MW - EMBED - HEADS), jnp.float32)], axis=1)
    m8 = _tr(w['eye8'], jnp.max(alpha, axis=0, keepdims=True))
    bmax = jnp.broadcast_to(m8, (HEADS, 128))

    @pl.when(i == 0)
    def _():
        cmax_o[:] = bmax

    @pl.when(i > 0)
    def _():
        cmax_o[:] = jnp.maximum(cmax_o[:], bmax)


def _weight_kernel(msg_b, cmax_b, wref, out_o):
    w = {k: r[...] for k, r in wref.items()}
    m = msg_b[:]
    cmax_row = _tr0(cmax_b[:, 0:1], w['eye8'])
    ae = jnp.exp(m[:, EMBED:EMBED + HEADS] - cmax_row)
    aerep = _mm(ae, w['rep8h'])
    out_o[:] = jnp.concatenate(
        [m[:, 0:EMBED] * aerep, ae,
         jnp.zeros((BE, MW - EMBED - HEADS), jnp.float32)], axis=1)


NHALF = NP // 2
NPH = NHALF + 8
GC_S = 80
GCH_S = GPW // GC_S
SCB = 400
SPG = SCB // GC_S
ZR = 160
RPS = NHALF // 16


def _scatter_body(msg_hbm, dst_hbm, out_hbm, idx_v, idxp_v, rows_v, zero_v,
                  acc_sh, sem):
    cid = lax.axis_index("c")
    sid = lax.axis_index("s")
    wid = sid * 2 + cid

    def zrow(j, _):
        for c in range(MW // 16):
            zero_v[j, pl.ds(c * 16, 16)] = jnp.zeros((16,), jnp.float32)
        return 0

    lax.fori_loop(0, ZR, zrow, 0)
    pltpu.sync_copy(dst_hbm.at[wid], idx_v)

    for p in range(2):
        for rpt in range(RPS // ZR):
            pltpu.sync_copy(zero_v,
                            acc_sh.at[pl.ds(sid * RPS + rpt * ZR, ZR)])

        def phase_idx(r, _):
            for c in range(GC_S // 16):
                sl = pl.ds(c * 16, 16)
                u = idx_v[r, sl] - p * NHALF
                ok = (u >= 0) & (u < NHALF)
                idxp_v[r, sl] = jnp.where(ok, u, NHALF)
            return 0

        lax.fori_loop(0, GCH_S, phase_idx, 0)
        plsc.subcore_barrier()

        def group(g, _):
            base = pl.multiple_of(wid * GPW + g * SCB, 8)
            pltpu.sync_copy(msg_hbm.at[pl.ds(base, SCB)], rows_v)
            for t in range(SPG):
                r = g * SPG + t
                pltpu.sync_copy(rows_v.at[pl.ds(t * GC_S, GC_S)],
                                acc_sh.at[idxp_v.at[r]], add=True)
            return 0

        lax.fori_loop(0, GPW // SCB, group, 0)
        plsc.subcore_barrier()
        dbase = pl.multiple_of(sid * RPS, 8)
        pltpu.sync_copy(acc_sh.at[pl.ds(dbase, RPS)],
                        out_hbm.at[cid, pl.ds(p * NHALF + dbase, RPS)])


def _sc_scatter(msgw, dst):
    dst3d = dst.reshape(GW, GCH_S, GC_S)
    return pl.kernel(
        _scatter_body,
        out_type=jax.ShapeDtypeStruct((2, NP, MW), jnp.float32),
        mesh=plsc.VectorSubcoreMesh(core_axis_name="c", subcore_axis_name="s"),
        scratch_types=[pltpu.VMEM((GCH_S, GC_S), jnp.int32),
                       pltpu.VMEM((GCH_S, GC_S), jnp.int32),
                       pltpu.VMEM((SCB, MW), jnp.float32),
                       pltpu.VMEM((ZR, MW), jnp.float32),
                       pltpu.VMEM_SHARED((NPH, MW), jnp.float32),
                       pltpu.SemaphoreType.DMA],
    )(msgw, dst3d)


def _final_kernel(acc_b, center_b, cn_b, wref, out_o):
    w = {k: r[...] for k, r in wref.items()}
    accs = acc_b[0]
    msum = accs[:, 0:EMBED]
    denom = accs[:, EMBED:EMBED + HEADS]
    denom_rep = _mm(denom, w['rep8h'])
    denom_rep = jnp.where(denom_rep == 0.0, 1.0, denom_rep)
    agg = msum / denom_rep
    cn = cn_b[:]
    center = center_b[:]
    gate = jax.nn.sigmoid(_mm(agg, w['ih_w']) + w['ih_b'] + _mm(cn, w['hh_w']) + w['hh_b'])
    upd = agg + gate * ((_mm(cn, w['self_w']) + w['self_b']) - agg)
    center = center + _mm(upd, w['out_w']) + w['out_b']
    cn2 = _ln(center, w['ln2_g'], w['ln2_b'])
    ff = _mm(jax.nn.relu(_mm(cn2, w['m_w1']) + w['m_b1']), w['m_w2']) + w['m_b2']
    out_o[:] = center + ff


def _full(ndim):
    return pl.BlockSpec(index_map=lambda *_: tuple(0 for _ in range(ndim)))


_WMAT = ['c_w1', 'c_w2', 'c_w3', 'nx_w1', 'nx_w2', 'ne_w1', 'ne_w2',
         'na_w', 'q_w', 'k_w', 'v_w', 'self_w', 'ih_w', 'hh_w',
         'out_w', 'm_w1', 'm_w2']
_WVEC = ['c_b1', 'c_b2', 'c_b3', 'nx_b1', 'nx_b2', 'ne_b1', 'ne_b2',
         'na_b', 'q_b', 'k_b', 'v_b', 'self_b', 'ih_b', 'hh_b',
         'out_b', 'm_b1', 'm_b2',
         'c_g1', 'c_be1', 'c_g2', 'c_be2', 'c_g3', 'c_be3', 'nx_g1',
         'nx_be1', 'ne_g1', 'ne_be1', 'na_g1', 'na_be1', 'na_g2',
         'na_be2', 'ln1_g', 'ln1_b', 'ln2_g', 'ln2_b']


def _prep_weights(p):
    w = {k: p[k].T for k in _WMAT}
    w.update({k: p[k].reshape(1, -1) for k in _WVEC})
    w['seg8'] = jnp.asarray(np.repeat(np.eye(8, dtype=np.float32), 8, axis=0))
    w['rep8h'] = jnp.asarray(np.repeat(np.eye(8, dtype=np.float32), 8, axis=1))
    w['eye8'] = jnp.asarray(np.eye(8, dtype=np.float32))
    return w


@jax.jit
def _run(x, edge_index, edge_attr, params):
    w = _prep_weights(params)
    xp = jnp.pad(x, ((0, NP - N), (0, 0)))
    src = edge_index[0].astype(jnp.int32)
    dst = edge_index[1].astype(jnp.int32)

    wspec = {k: _full(jnp.ndim(v)) for k, v in w.items()}

    p1_keys = ['c_w1', 'c_b1', 'c_g1', 'c_be1', 'c_w2', 'c_b2', 'c_g2', 'c_be2',
               'c_w3', 'c_b3', 'c_g3', 'c_be3', 'ln1_g', 'ln1_b',
               'nx_w1', 'nx_b1', 'nx_g1', 'nx_be1', 'nx_w2', 'nx_b2',
               'q_w', 'q_b']
    center, cn, hxq = pl.pallas_call(
        _node_kernel,
        grid=(NB,),
        in_specs=[pl.BlockSpec((BN, NODE_DIM), lambda i: (i, 0)),
                  {k: wspec[k] for k in p1_keys}],
        out_specs=[pl.BlockSpec((BN, EMBED), lambda i: (i, 0)),
                   pl.BlockSpec((BN, EMBED), lambda i: (i, 0)),
                   pl.BlockSpec((BN, 2 * EMBED), lambda i: (i, 0))],
        out_shape=[jax.ShapeDtypeStruct((NP, EMBED), jnp.float32),
                   jax.ShapeDtypeStruct((NP, EMBED), jnp.float32),
                   jax.ShapeDtypeStruct((NP, 2 * EMBED), jnp.float32)],
    )(xp, {k: w[k] for k in p1_keys})

    hx_e, qe = _sc_gather(hxq, src, dst)

    p3_keys = ['ne_w1', 'ne_b1', 'ne_g1', 'ne_be1', 'ne_w2', 'ne_b2',
               'na_g1', 'na_be1', 'na_w', 'na_b', 'na_g2', 'na_be2',
               'k_w', 'k_b', 'v_w', 'v_b', 'seg8', 'eye8']
    msg_u, cmax = pl.pallas_call(
        _edge_kernel,
        grid=(EB,),
        in_specs=[pl.BlockSpec((BE, 2 * EMBED), lambda i: (i, 0)),
                  pl.BlockSpec((BE, 2 * EMBED), lambda i: (i, 0)),
                  pl.BlockSpec((BE, EDGE_DIM), lambda i: (i, 0)),
                  {k: wspec[k] for k in p3_keys}],
        out_specs=[pl.BlockSpec((BE, MW), lambda i: (i, 0)),
                   pl.BlockSpec((HEADS, 128), lambda i: (0, 0))],
        out_shape=[jax.ShapeDtypeStruct((E, MW), jnp.float32),
                   jax.ShapeDtypeStruct((HEADS, 128), jnp.float32)],
    )(hx_e, qe, edge_attr, {k: w[k] for k in p3_keys})

    p4_keys = ['eye8', 'rep8h']
    msgw = pl.pallas_call(
        _weight_kernel,
        grid=(EB,),
        in_specs=[pl.BlockSpec((BE, MW), lambda i: (i, 0)),
                  pl.BlockSpec((HEADS, 128), lambda i: (0, 0)),
                  {k: wspec[k] for k in p4_keys}],
        out_specs=pl.BlockSpec((BE, MW), lambda i: (i, 0)),
        out_shape=jax.ShapeDtypeStruct((E, MW), jnp.float32),
    )(msg_u, cmax, {k: w[k] for k in p4_keys})

    acc = jax.ops.segment_sum(msgw, dst, num_segments=NP)[None]

    p5_keys = ['ih_w', 'ih_b', 'hh_w', 'hh_b', 'self_w', 'self_b',
               'out_w', 'out_b', 'ln2_g', 'ln2_b',
               'm_w1', 'm_b1', 'm_w2', 'm_b2', 'rep8h']
    out = pl.pallas_call(
        _final_kernel,
        grid=(NB,),
        in_specs=[pl.BlockSpec((1, BN, MW), lambda i: (0, i, 0)),
                  pl.BlockSpec((BN, EMBED), lambda i: (i, 0)),
                  pl.BlockSpec((BN, EMBED), lambda i: (i, 0)),
                  {k: wspec[k] for k in p5_keys}],
        out_specs=pl.BlockSpec((BN, EMBED), lambda i: (i, 0)),
        out_shape=jax.ShapeDtypeStruct((NP, EMBED), jnp.float32),
    )(acc, center, cn, {k: w[k] for k in p5_keys})
    return out[:N]


def kernel(x, edge_index, edge_attr, params):
    return _run(x, edge_index, edge_attr, params)
